# baseline (speedup 1.0000x reference)
import jax
import jax.numpy as jnp
from jax import lax
from jax.experimental import pallas as pl
from jax.experimental.pallas import tpu as pltpu, tpu_sc as plsc

B = 16384
V = 100000
D = 128
P = 20
N = 50
C = P + N

NC = 2
NS = 16
NW = NC * NS
BPW = B // NW
SR = 256
NSC = BPW // SR
E = SR * C
L = 16


NB = 4
CP = (C + L - 1) // L * L
NG = CP // L


def _w2v_body(cids_hbm, wid_hbm, wtab_hbm, ctab_hbm, out_hbm,
              cids_v, widx_v, wrows_v, crows_v, dots_v, wsem, sems):
    wid = lax.axis_index("s") * NC + lax.axis_index("c")
    base = wid * BPW

    def gather_row(r, b):
        return pltpu.make_async_copy(
            ctab_hbm.at[cids_v.at[r]], crows_v.at[b, pl.ds(0, C), :], sems.at[b]
        )

    def superchunk(sc, _):
        row0 = base + sc * SR
        pltpu.sync_copy(wid_hbm.at[pl.ds(row0, SR)], widx_v)
        pltpu.sync_copy(cids_hbm.at[pl.ds(row0, SR), :], cids_v)
        wcopy = pltpu.make_async_copy(wtab_hbm.at[widx_v], wrows_v, wsem)
        wcopy.start()
        for b in range(NB - 1):
            gather_row(b, b).start()
        wcopy.wait()

        lane = lax.iota(jnp.int32, L)

        def step(r, _):
            b = lax.rem(r, NB)
            gather_row(r, b).wait()

            @pl.when(r + NB - 1 < SR)
            def _():
                nxt = r + NB - 1
                gather_row(nxt, lax.rem(nxt, NB)).start()

            wv = [wrows_v[r, pl.ds(k * L, L)] for k in range(D // L)]
            for g in range(NG):
                gvec = jnp.zeros((L,), jnp.float32)
                for i in range(L):
                    j = g * L + i
                    p = [crows_v[b, j, pl.ds(k * L, L)] * wv[k]
                         for k in range(D // L)]
                    while len(p) > 1:
                        p = [p[z] + p[z + 1] for z in range(0, len(p), 2)]
                    gvec = jnp.where(lane == i, jnp.sum(p[0]), gvec)
                sig = 1.0 / (1.0 + jnp.exp(-gvec))
                dots_v[pl.ds(r * C + g * L, L)] = sig
            return 0

        lax.fori_loop(0, SR, step, 0)
        pltpu.sync_copy(dots_v.at[pl.ds(0, E)], out_hbm.at[pl.ds(row0 * C, E)])
        return 0

    lax.fori_loop(0, NSC, superchunk, 0)


def kernel(word_id, positive_context_ids, negative_context_ids, W_word, W_ctx):
    ctx_ids = jnp.concatenate(
        [positive_context_ids, negative_context_ids], axis=1
    ).astype(jnp.int32)
    wid32 = word_id.astype(jnp.int32)

    mesh = plsc.VectorSubcoreMesh(core_axis_name="c", subcore_axis_name="s")
    run = pl.kernel(
        _w2v_body,
        out_type=jax.ShapeDtypeStruct((B * C,), jnp.float32),
        mesh=mesh,
        compiler_params=pltpu.CompilerParams(needs_layout_passes=False),
        scratch_types=[
            pltpu.VMEM((SR, C), jnp.int32),
            pltpu.VMEM((SR,), jnp.int32),
            pltpu.VMEM((SR, D), jnp.float32),
            pltpu.VMEM((NB, CP, D), jnp.float32),
            pltpu.VMEM((E + L,), jnp.float32),
            pltpu.SemaphoreType.DMA,
            pltpu.SemaphoreType.DMA((NB,)),
        ],
    )
    out = run(ctx_ids, wid32, W_word, W_ctx).reshape(B, C)
    return out[:, :P], out[:, P:]


# speedup vs baseline: 1.0015x; 1.0015x over previous
import jax
import jax.numpy as jnp
from jax import lax
from jax.experimental import pallas as pl
from jax.experimental.pallas import tpu as pltpu, tpu_sc as plsc

B = 16384
V = 100000
D = 128
P = 20
N = 50
C = P + N

NC = 2
NS = 16
NW = NC * NS
BPW = B // NW
SR = 256
NSC = BPW // SR
E = SR * C
L = 16


NB = 4
CP = (C + L - 1) // L * L
NG = CP // L


def _w2v_body(cids_hbm, wid_hbm, wtab_hbm, ctab_hbm, out_hbm,
              cids_v, widx_v, wrows_v, crows_v, dots_v, wsem, sems):
    wid = lax.axis_index("s") * NC + lax.axis_index("c")
    base = wid * BPW

    def gather_row(r, b):
        return pltpu.make_async_copy(
            ctab_hbm.at[cids_v.at[r]], crows_v.at[b, pl.ds(0, C), :], sems.at[b]
        )

    def superchunk(sc, _):
        row0 = base + sc * SR
        pltpu.sync_copy(wid_hbm.at[pl.ds(row0, SR)], widx_v)
        pltpu.sync_copy(cids_hbm.at[pl.ds(row0, SR), :], cids_v)
        wcopies = [
            pltpu.make_async_copy(
                wtab_hbm.at[widx_v.at[pl.ds(c * 128, 128)]],
                wrows_v.at[pl.ds(c * 128, 128), :], wsem)
            for c in range(SR // 128)
        ]
        for wc in wcopies:
            wc.start()
        for b in range(NB - 1):
            gather_row(b, b).start()
        for wc in wcopies:
            wc.wait()

        lane = lax.iota(jnp.int32, L)

        def step(r, _):
            b = lax.rem(r, NB)
            gather_row(r, b).wait()

            @pl.when(r + NB - 1 < SR)
            def _():
                nxt = r + NB - 1
                gather_row(nxt, lax.rem(nxt, NB)).start()

            wv = [wrows_v[r, pl.ds(k * L, L)] for k in range(D // L)]
            for g in range(NG):
                gvec = jnp.zeros((L,), jnp.float32)
                for i in range(L):
                    j = g * L + i
                    p = [crows_v[b, j, pl.ds(k * L, L)] * wv[k]
                         for k in range(D // L)]
                    while len(p) > 1:
                        p = [p[z] + p[z + 1] for z in range(0, len(p), 2)]
                    gvec = jnp.where(lane == i, jnp.sum(p[0]), gvec)
                sig = 1.0 / (1.0 + jnp.exp(-gvec))
                dots_v[pl.ds(r * C + g * L, L)] = sig
            return 0

        lax.fori_loop(0, SR, step, 0)
        pltpu.sync_copy(dots_v.at[pl.ds(0, E)], out_hbm.at[pl.ds(row0 * C, E)])
        return 0

    lax.fori_loop(0, NSC, superchunk, 0)


def kernel(word_id, positive_context_ids, negative_context_ids, W_word, W_ctx):
    ctx_ids = jnp.concatenate(
        [positive_context_ids, negative_context_ids], axis=1
    ).astype(jnp.int32)
    wid32 = word_id.astype(jnp.int32)

    mesh = plsc.VectorSubcoreMesh(core_axis_name="c", subcore_axis_name="s")
    run = pl.kernel(
        _w2v_body,
        out_type=jax.ShapeDtypeStruct((B * C,), jnp.float32),
        mesh=mesh,
        compiler_params=pltpu.CompilerParams(needs_layout_passes=False),
        scratch_types=[
            pltpu.VMEM((SR, C), jnp.int32),
            pltpu.VMEM((SR,), jnp.int32),
            pltpu.VMEM((SR, D), jnp.float32),
            pltpu.VMEM((NB, CP, D), jnp.float32),
            pltpu.VMEM((E + L,), jnp.float32),
            pltpu.SemaphoreType.DMA,
            pltpu.SemaphoreType.DMA((NB,)),
        ],
    )
    out = run(ctx_ids, wid32, W_word, W_ctx).reshape(B, C)
    return out[:, :P], out[:, P:]
